# baseline (speedup 1.0000x reference)
import functools

import jax
import jax.numpy as jnp
from jax import lax
from jax.experimental import pallas as pl
from jax.experimental.pallas import tpu as pltpu
from jax.experimental.pallas import tpu_sc as plsc

N = 10000
E = 320000
D = 128
L = 16
NC = 2
NS = 16
NW = NC * NS
NP = 10240
B = 80
NB = 128
EPW = NB * B
EP = NW * EPW
G = 8
NG = NB // G
NBUF_CNT = 3
NBUF_NOCNT = 4
RPT = NP // NS
RB = 640


def _sc_agg_body(with_count, table, src, dst, acc_out, cnt_out, acc_sh,
                 rows, sidx, didx, cntp, sems):
    c = lax.axis_index("c")
    s = lax.axis_index("s")
    wid = s * NC + c

    def zero_row(r, carry):
        for j in range(D // L):
            rows[0][r, pl.ds(j * L, L)] = jnp.zeros((L,), jnp.float32)
        return carry
    lax.fori_loop(0, B, zero_row, 0)
    t0 = pl.multiple_of(s * RPT, 8)
    for k in range(RPT // B):
        pltpu.sync_copy(rows[0], acc_sh.at[pl.ds(t0 + k * B, B)])
    if with_count:
        zl = jnp.zeros((L,), jnp.float32)
        for r in range(NP // L):
            cntp[pl.ds(r * L, L)] = zl
    plsc.subcore_barrier()

    g0 = wid * NG
    ones_v = jnp.ones((L,), jnp.float32)
    nbuf = len(rows)

    def scat(b, j):
        pltpu.sync_copy(rows[b], acc_sh.at[didx.at[j]], add=True)
        if with_count:
            for jj in range(B // L):
                dv = didx[j, pl.ds(jj * L, L)]
                plsc.addupdate_scatter(cntp, [dv], ones_v)

    def step(k, carry):
        pltpu.sync_copy(src.at[g0 + k], sidx)
        pltpu.sync_copy(dst.at[g0 + k], didx)
        gs = [pltpu.async_copy(table.at[sidx.at[b]], rows[b], sems[b])
              for b in range(nbuf)]
        for j in range(G):
            b = j % nbuf
            gs[b].wait()
            scat(b, j)
            nj = j + nbuf
            if nj < G:
                gs[b] = pltpu.async_copy(
                    table.at[sidx.at[nj]], rows[b], sems[b])
        return carry
    lax.fori_loop(0, NG, step, 0)
    plsc.subcore_barrier()

    o0 = pl.multiple_of(c * NP + t0, 8)
    pltpu.sync_copy(acc_sh.at[pl.ds(t0, RPT)], acc_out.at[pl.ds(o0, RPT)])
    if with_count:
        pltpu.sync_copy(cntp, cnt_out.at[wid])


def _make_sc_agg(with_count):
    mesh = plsc.VectorSubcoreMesh(
        core_axis_name="c", subcore_axis_name="s", num_cores=NC, num_subcores=NS)
    out_type = [jax.ShapeDtypeStruct((NC * NP, D), jnp.float32)]
    if with_count:
        out_type.append(jax.ShapeDtypeStruct((NW, NP), jnp.float32))
    nbuf = NBUF_CNT if with_count else NBUF_NOCNT
    scratch = (
        [pltpu.VMEM_SHARED((NP, D), jnp.float32)]
        + [pltpu.VMEM((B, D), jnp.float32)] * nbuf
        + [pltpu.VMEM((G, B), jnp.int32)]
        + [pltpu.VMEM((G, B), jnp.int32)]
        + [pltpu.VMEM((NP if with_count else L,), jnp.float32)]
        + [pltpu.SemaphoreType.DMA] * nbuf
    )

    def body(table, src, dst, acc_out, *rest):
        if with_count:
            cnt_out = rest[0]
            rest = rest[1:]
        else:
            cnt_out = None
        acc_sh = rest[0]
        rows = list(rest[1:1 + nbuf])
        sidx = rest[1 + nbuf]
        didx = rest[2 + nbuf]
        cntp = rest[3 + nbuf]
        sems = list(rest[4 + nbuf:4 + 2 * nbuf])
        _sc_agg_body(with_count, table, src, dst, acc_out, cnt_out,
                     acc_sh, rows, sidx, didx, cntp, sems)

    return pl.kernel(
        body, out_type=out_type, mesh=mesh, scratch_types=scratch,
        compiler_params=pltpu.CompilerParams(needs_layout_passes=False))


_sc_agg_cnt = _make_sc_agg(True)
_sc_agg = _make_sc_agg(False)


def _tc_linear_body(relu, a0, a1, cnt, xr, wl, wr, br, out):
    csum = jnp.sum(cnt[...], axis=0)
    inv = 1.0 / jnp.maximum(csum, 1.0)
    mean = (a0[...] + a1[...]) * inv[:, None]
    y = jnp.dot(mean, wl[...], preferred_element_type=jnp.float32,
                precision=lax.Precision.HIGHEST)
    y = y + jnp.dot(xr[...], wr[...], preferred_element_type=jnp.float32,
                    precision=lax.Precision.HIGHEST)
    y = y + br[...]
    out[...] = jnp.maximum(y, 0.0) if relu else y


def _tc_linear(acc0, acc1, cnt, x, wlT, wrT, b, relu):
    blk = lambda r, c: pl.BlockSpec((r, c), lambda i: (i, 0))
    full = lambda r, c: pl.BlockSpec((r, c), lambda i: (0, 0))
    return pl.pallas_call(
        functools.partial(_tc_linear_body, relu),
        grid=(NP // RB,),
        in_specs=[blk(RB, D), blk(RB, D),
                  pl.BlockSpec((NW, RB), lambda i: (0, i)), blk(RB, D),
                  full(D, D), full(D, D), full(1, D)],
        out_specs=blk(RB, D),
        out_shape=jax.ShapeDtypeStruct((NP, D), jnp.float32),
    )(acc0, acc1, cnt, x, wlT, wrT, b)


def kernel(x, edge_index, W1_l, W1_r, b1, W2_l, W2_r, b2):
    src = jnp.concatenate([edge_index[0].astype(jnp.int32),
                           jnp.zeros((EP - E,), jnp.int32)]).reshape(NW * NG, G, B)
    dst = jnp.concatenate([edge_index[1].astype(jnp.int32),
                           jnp.full((EP - E,), NP - 1, jnp.int32)]).reshape(NW * NG, G, B)
    xp = jnp.concatenate([x, jnp.zeros((NP - N, D), jnp.float32)], axis=0)
    acc, cnt = _sc_agg_cnt(xp, src, dst)
    h = _tc_linear(acc[:NP], acc[NP:], cnt, xp,
                   W1_l.T, W1_r.T, b1[None, :], True)
    acc2, = _sc_agg(h, src, dst)
    out = _tc_linear(acc2[:NP], acc2[NP:], cnt, h,
                     W2_l.T, W2_r.T, b2[None, :], False)
    return out[:N]


# speedup vs baseline: 2.3456x; 2.3456x over previous
import functools

import jax
import jax.numpy as jnp
from jax import lax
from jax.experimental import pallas as pl
from jax.experimental.pallas import tpu as pltpu
from jax.experimental.pallas import tpu_sc as plsc

N = 10000
E = 320000
D = 128
L = 16
NC = 2
NS = 16
NW = NC * NS
NP = 10240
B = 80
EPW = E // NW
NB = EPW // B
NBUF_CNT = 3
NBUF_NOCNT = 4
RPT = NP // NS
RB = 640


def _sc_agg_body(with_count, table, src, dst, acc_out, cnt_out, acc_sh,
                 rows, sidx, didx, cntp, sems):
    c = lax.axis_index("c")
    s = lax.axis_index("s")
    wid = s * NC + c

    def zero_row(r, carry):
        for j in range(D // L):
            rows[0][r, pl.ds(j * L, L)] = jnp.zeros((L,), jnp.float32)
        return carry
    lax.fori_loop(0, B, zero_row, 0)
    t0 = pl.multiple_of(s * RPT, 8)
    for k in range(RPT // B):
        pltpu.sync_copy(rows[0], acc_sh.at[pl.ds(t0 + k * B, B)])
    if with_count:
        zl = jnp.zeros((L,), jnp.float32)
        for r in range(NP // L):
            cntp[pl.ds(r * L, L)] = zl
    plsc.subcore_barrier()

    e0 = wid * EPW
    ones_v = jnp.ones((L,), jnp.float32)
    nbuf = len(rows)

    def load_idx(b, base):
        pltpu.sync_copy(src.at[pl.ds(base, B)], sidx[b])
        pltpu.sync_copy(dst.at[pl.ds(base, B)], didx[b])

    def scat(b):
        pltpu.sync_copy(rows[b], acc_sh.at[didx[b]], add=True)
        if with_count:
            for j in range(B // L):
                dv = didx[b][pl.ds(j * L, L)]
                plsc.addupdate_scatter(cntp, [dv], ones_v)

    def step(k, carry):
        gs = []
        for b in range(nbuf):
            load_idx(b, pl.multiple_of(e0 + (nbuf * k + b) * B, 8))
            gs.append(pltpu.async_copy(table.at[sidx[b]], rows[b], sems[b]))
        for b in range(nbuf):
            gs[b].wait()
            scat(b)
        return carry
    lax.fori_loop(0, NB // nbuf, step, 0)
    for t in range(NB - (NB // nbuf) * nbuf):
        load_idx(0, pl.multiple_of(e0 + ((NB // nbuf) * nbuf + t) * B, 8))
        pltpu.async_copy(table.at[sidx[0]], rows[0], sems[0]).wait()
        scat(0)
    plsc.subcore_barrier()

    o0 = pl.multiple_of(c * NP + t0, 8)
    pltpu.sync_copy(acc_sh.at[pl.ds(t0, RPT)], acc_out.at[pl.ds(o0, RPT)])
    if with_count:
        pltpu.sync_copy(cntp, cnt_out.at[wid])


def _make_sc_agg(with_count):
    mesh = plsc.VectorSubcoreMesh(
        core_axis_name="c", subcore_axis_name="s", num_cores=NC, num_subcores=NS)
    out_type = [jax.ShapeDtypeStruct((NC * NP, D), jnp.float32)]
    if with_count:
        out_type.append(jax.ShapeDtypeStruct((NW, NP), jnp.float32))
    nbuf = NBUF_CNT if with_count else NBUF_NOCNT
    scratch = (
        [pltpu.VMEM_SHARED((NP, D), jnp.float32)]
        + [pltpu.VMEM((B, D), jnp.float32)] * nbuf
        + [pltpu.VMEM((B,), jnp.int32)] * nbuf
        + [pltpu.VMEM((B,), jnp.int32)] * nbuf
        + [pltpu.VMEM((NP if with_count else L,), jnp.float32)]
        + [pltpu.SemaphoreType.DMA] * nbuf
    )

    def body(table, src, dst, acc_out, *rest):
        if with_count:
            cnt_out = rest[0]
            rest = rest[1:]
        else:
            cnt_out = None
        acc_sh = rest[0]
        rows = list(rest[1:1 + nbuf])
        sidx = list(rest[1 + nbuf:1 + 2 * nbuf])
        didx = list(rest[1 + 2 * nbuf:1 + 3 * nbuf])
        cntp = rest[1 + 3 * nbuf]
        sems = list(rest[2 + 3 * nbuf:2 + 4 * nbuf])
        _sc_agg_body(with_count, table, src, dst, acc_out, cnt_out,
                     acc_sh, rows, sidx, didx, cntp, sems)

    return pl.kernel(
        body, out_type=out_type, mesh=mesh, scratch_types=scratch,
        compiler_params=pltpu.CompilerParams(needs_layout_passes=False))


_sc_agg_cnt = _make_sc_agg(True)
_sc_agg = _make_sc_agg(False)


def _tc_linear_body(relu, a0, a1, cnt, xr, wl, wr, br, out):
    csum = jnp.sum(cnt[...], axis=0)
    inv = 1.0 / jnp.maximum(csum, 1.0)
    mean = (a0[...] + a1[...]) * inv[:, None]
    y = jnp.dot(mean, wl[...], preferred_element_type=jnp.float32,
                precision=lax.Precision.HIGHEST)
    y = y + jnp.dot(xr[...], wr[...], preferred_element_type=jnp.float32,
                    precision=lax.Precision.HIGHEST)
    y = y + br[...]
    out[...] = jnp.maximum(y, 0.0) if relu else y


def _tc_linear(acc0, acc1, cnt, x, wlT, wrT, b, relu):
    blk = lambda r, c: pl.BlockSpec((r, c), lambda i: (i, 0))
    full = lambda r, c: pl.BlockSpec((r, c), lambda i: (0, 0))
    return pl.pallas_call(
        functools.partial(_tc_linear_body, relu),
        grid=(NP // RB,),
        in_specs=[blk(RB, D), blk(RB, D),
                  pl.BlockSpec((NW, RB), lambda i: (0, i)), blk(RB, D),
                  full(D, D), full(D, D), full(1, D)],
        out_specs=blk(RB, D),
        out_shape=jax.ShapeDtypeStruct((NP, D), jnp.float32),
    )(acc0, acc1, cnt, x, wlT, wrT, b)


def kernel(x, edge_index, W1_l, W1_r, b1, W2_l, W2_r, b2):
    src = edge_index[0].astype(jnp.int32)
    dst = edge_index[1].astype(jnp.int32)
    xp = jnp.concatenate([x, jnp.zeros((NP - N, D), jnp.float32)], axis=0)
    acc, cnt = _sc_agg_cnt(xp, src, dst)
    h = _tc_linear(acc[:NP], acc[NP:], cnt, xp,
                   W1_l.T, W1_r.T, b1[None, :], True)
    acc2, = _sc_agg(h, src, dst)
    out = _tc_linear(acc2[:NP], acc2[NP:], cnt, h,
                     W2_l.T, W2_r.T, b2[None, :], False)
    return out[:N]
